# Initial kernel scaffold; baseline (speedup 1.0000x reference)
import functools

import jax
import jax.numpy as jnp
from jax import lax
from jax.experimental import pallas as pl
from jax.experimental.pallas import tpu as pltpu
from jax.experimental.pallas import tpu_sc as plsc

N = 10000
E = 160000
D = 256
DH = 128
B = 512
EC = 128
NECHUNK = E // EC
PC = 80
NPCHUNK = N // PC
NSUB = 16
RB = 1000

_mesh = plsc.VectorSubcoreMesh(core_axis_name="c", subcore_axis_name="s")
_f32 = jnp.float32


def _mm_first_body(h_ref, wn_ref, wr_ref, b_ref, m0_ref, m1_ref, r_ref):
    h = h_ref[...]
    m = jnp.dot(h, wn_ref[...], preferred_element_type=jnp.float32)
    m0_ref[...] = m[:, :DH]
    m1_ref[...] = m[:, DH:]
    r_ref[...] = jnp.dot(h, wr_ref[...], preferred_element_type=jnp.float32) + b_ref[...]


def _mm_fused_body(a0_ref, a1_ref, rp_ref, wn_ref, wr_ref, b_ref, m0_ref, m1_ref, r_ref):
    h = jnp.maximum(
        jnp.concatenate([a0_ref[...], a1_ref[...]], axis=1) + rp_ref[...], 0.0)
    m = jnp.dot(h, wn_ref[...], preferred_element_type=jnp.float32)
    m0_ref[...] = m[:, :DH]
    m1_ref[...] = m[:, DH:]
    r_ref[...] = jnp.dot(h, wr_ref[...], preferred_element_type=jnp.float32) + b_ref[...]


def _relu_body(a0_ref, a1_ref, rp_ref, h_ref):
    h_ref[...] = jnp.maximum(
        jnp.concatenate([a0_ref[...], a1_ref[...]], axis=1) + rp_ref[...], 0.0)


def _row_spec(w):
    return pl.BlockSpec((RB, w), lambda i: (i, 0))


def _full_spec(a, b):
    return pl.BlockSpec((a, b), lambda i: (0, 0))


_mm_first = pl.pallas_call(
    _mm_first_body,
    grid=(N // RB,),
    in_specs=[_row_spec(D), _full_spec(D, D), _full_spec(D, D), _full_spec(1, D)],
    out_specs=[_row_spec(DH), _row_spec(DH), _row_spec(D)],
    out_shape=[jax.ShapeDtypeStruct((N, DH), _f32),
               jax.ShapeDtypeStruct((N, DH), _f32),
               jax.ShapeDtypeStruct((N, D), _f32)],
)

_mm_fused = pl.pallas_call(
    _mm_fused_body,
    grid=(N // RB,),
    in_specs=[_row_spec(DH), _row_spec(DH), _row_spec(D),
              _full_spec(D, D), _full_spec(D, D), _full_spec(1, D)],
    out_specs=[_row_spec(DH), _row_spec(DH), _row_spec(D)],
    out_shape=[jax.ShapeDtypeStruct((N, DH), _f32),
               jax.ShapeDtypeStruct((N, DH), _f32),
               jax.ShapeDtypeStruct((N, D), _f32)],
)

_relu = pl.pallas_call(
    _relu_body,
    grid=(N // RB,),
    in_specs=[_row_spec(DH), _row_spec(DH), _row_spec(D)],
    out_specs=_row_spec(D),
    out_shape=jax.ShapeDtypeStruct((N, D), _f32),
)


def _head_body(p0s, p1s, cs, rs, p0t, p1t, ct, rt, wlin, blin, wc, bc, lab,
               logits_ref, loss_ref):
    def emb(p0, p1, cnt, root):
        inv = 1.0 / jnp.maximum(cnt[...], 1.0)
        z = jnp.concatenate([p0[...] * inv, p1[...] * inv, root[...]], axis=1)
        return jnp.dot(z, wlin[...], preferred_element_type=jnp.float32) + blin[...]

    u = emb(p0s, p1s, cs, rs)
    v = emb(p0t, p1t, ct, rt)
    feats = jnp.concatenate([u, v, jnp.abs(u - v)], axis=1)
    logits = jnp.dot(feats, wc[...], preferred_element_type=jnp.float32) + bc[...]
    logits_ref[...] = logits
    mx = jnp.max(logits, axis=1, keepdims=True)
    lse = mx + jnp.log(jnp.sum(jnp.exp(logits - mx), axis=1, keepdims=True))
    logp = logits - lse
    sel = jnp.where(lab[...] == 0, logp[:, :1], logp[:, 1:2])
    loss_ref[...] = jnp.full((1, 1), -1.0 / B, jnp.float32) * jnp.sum(sel)


_head = pl.pallas_call(
    _head_body,
    in_specs=[pl.BlockSpec(memory_space=pltpu.VMEM)] * 13,
    out_specs=[pl.BlockSpec(memory_space=pltpu.VMEM)] * 2,
    out_shape=[jax.ShapeDtypeStruct((B, 2), _f32),
               jax.ShapeDtypeStruct((1, 1), _f32)],
)


_ROWS_PER_SUB = N // NSUB


@functools.partial(
    pl.kernel, mesh=_mesh,
    out_type=(jax.ShapeDtypeStruct((N, DH), _f32),
              jax.ShapeDtypeStruct((N, DH), _f32)),
    scratch_types=[
        pltpu.VMEM((EC,), jnp.int32),
        pltpu.VMEM((EC,), jnp.int32),
        pltpu.VMEM((EC, DH), _f32),
        pltpu.VMEM_SHARED((N, DH), _f32),
        pltpu.SemaphoreType.DMA,
    ])
def _scatter(m0_hbm, m1_hbm, src_hbm, dst_hbm, zero_hbm, a0_hbm, a1_hbm,
             idx_s, idx_d, rows, acc, sem):
    c = lax.axis_index("c")
    s = lax.axis_index("s")
    r0 = s * _ROWS_PER_SUB
    pltpu.sync_copy(zero_hbm.at[pl.ds(r0, _ROWS_PER_SUB)],
                    acc.at[pl.ds(r0, _ROWS_PER_SUB)])
    plsc.subcore_barrier()

    def edge_loop(m_hbm):
        @pl.loop(s, NECHUNK, step=NSUB)
        def _(k):
            base = k * EC
            pltpu.sync_copy(src_hbm.at[pl.ds(base, EC)], idx_s)
            pltpu.sync_copy(dst_hbm.at[pl.ds(base, EC)], idx_d)
            pltpu.async_copy(m_hbm.at[idx_s], rows, sem).wait()
            pltpu.sync_copy(rows, acc.at[idx_d], add=True)

    @pl.when(c == 0)
    def _():
        edge_loop(m0_hbm)

    @pl.when(c == 1)
    def _():
        edge_loop(m1_hbm)

    plsc.subcore_barrier()

    @pl.when(c == 0)
    def _():
        pltpu.sync_copy(acc.at[pl.ds(r0, _ROWS_PER_SUB)],
                        a0_hbm.at[pl.ds(r0, _ROWS_PER_SUB)])

    @pl.when(c == 1)
    def _():
        pltpu.sync_copy(acc.at[pl.ds(r0, _ROWS_PER_SUB)],
                        a1_hbm.at[pl.ds(r0, _ROWS_PER_SUB)])


_BROWS_PER_SUB = B // NSUB


@functools.partial(
    pl.kernel, mesh=_mesh,
    out_type=(jax.ShapeDtypeStruct((B, DH), _f32),
              jax.ShapeDtypeStruct((B, DH), _f32),
              jax.ShapeDtypeStruct((B, DH), _f32),
              jax.ShapeDtypeStruct((B, D), _f32)),
    scratch_types=[
        pltpu.VMEM((PC,), jnp.int32),
        pltpu.VMEM((PC, DH), _f32),
        pltpu.VMEM((PC, DH), _f32),
        pltpu.VMEM((16,), jnp.int32),
        pltpu.VMEM((16, D), _f32),
        pltpu.VMEM_SHARED((B, DH), _f32),
        pltpu.VMEM_SHARED((B, DH), _f32),
        pltpu.SemaphoreType.DMA,
    ])
def _pool(h_hbm, batch_hbm, root_hbm, ones_hbm, zero_hbm,
          p0_hbm, p1_hbm, cnt_hbm, remb_hbm,
          idx_v, rows_v, ones_v, ridx_v, rrows_v, acc, cacc, sem):
    c = lax.axis_index("c")
    s = lax.axis_index("s")
    b0 = s * _BROWS_PER_SUB
    pltpu.sync_copy(zero_hbm.at[pl.ds(b0, _BROWS_PER_SUB)],
                    acc.at[pl.ds(b0, _BROWS_PER_SUB)])
    pltpu.sync_copy(zero_hbm.at[pl.ds(b0, _BROWS_PER_SUB)],
                    cacc.at[pl.ds(b0, _BROWS_PER_SUB)])
    pltpu.sync_copy(ones_hbm, ones_v)
    plsc.subcore_barrier()

    w = s * 2 + c
    pltpu.sync_copy(root_hbm.at[pl.ds(w * 16, 16)], ridx_v)
    pltpu.async_copy(h_hbm.at[ridx_v], rrows_v, sem).wait()
    pltpu.sync_copy(rrows_v, remb_hbm.at[pl.ds(w * 16, 16)])

    def pool_loop(col):
        @pl.loop(s, NPCHUNK, step=NSUB)
        def _(k):
            base = k * PC
            pltpu.sync_copy(batch_hbm.at[pl.ds(base, PC)], idx_v)
            pltpu.sync_copy(h_hbm.at[pl.ds(base, PC), pl.ds(col, DH)], rows_v)
            pltpu.sync_copy(rows_v, acc.at[idx_v], add=True)
            pltpu.sync_copy(ones_v, cacc.at[idx_v], add=True)

    @pl.when(c == 0)
    def _():
        pool_loop(0)

    @pl.when(c == 1)
    def _():
        pool_loop(DH)

    plsc.subcore_barrier()

    @pl.when(c == 0)
    def _():
        pltpu.sync_copy(acc.at[pl.ds(b0, _BROWS_PER_SUB)],
                        p0_hbm.at[pl.ds(b0, _BROWS_PER_SUB)])
        pltpu.sync_copy(cacc.at[pl.ds(b0, _BROWS_PER_SUB)],
                        cnt_hbm.at[pl.ds(b0, _BROWS_PER_SUB)])

    @pl.when(c == 1)
    def _():
        pltpu.sync_copy(acc.at[pl.ds(b0, _BROWS_PER_SUB)],
                        p1_hbm.at[pl.ds(b0, _BROWS_PER_SUB)])


def kernel(x_s, edge_index_s, batch_s, root_n_id_s, x_t, edge_index_t,
           batch_t, root_n_id_t, labels, Wr1, Wn1, b1, Wr2, Wn2, b2,
           Wr3, Wn3, b3, Wlin, blin, Wc, bc):
    zeros = jnp.zeros((N, DH), _f32)
    ones = jnp.ones((PC, DH), _f32)
    b1r = b1.reshape(1, D)
    b2r = b2.reshape(1, D)
    b3r = b3.reshape(1, D)

    def gnn(x, ei, batch, root):
        src, dst = ei[0], ei[1]
        m0, m1, r = _mm_first(x, Wn1, Wr1, b1r)
        a0, a1 = _scatter(m0, m1, src, dst, zeros)
        m0, m1, r = _mm_fused(a0, a1, r, Wn2, Wr2, b2r)
        a0, a1 = _scatter(m0, m1, src, dst, zeros)
        m0, m1, r = _mm_fused(a0, a1, r, Wn3, Wr3, b3r)
        a0, a1 = _scatter(m0, m1, src, dst, zeros)
        h3 = _relu(a0, a1, r)
        return _pool(h3, batch, root, ones, zeros)

    p0s, p1s, cs, rs = gnn(x_s, edge_index_s, batch_s, root_n_id_s)
    p0t, p1t, ct, rt = gnn(x_t, edge_index_t, batch_t, root_n_id_t)

    logits, loss11 = _head(p0s, p1s, cs, rs, p0t, p1t, ct, rt,
                           Wlin, blin.reshape(1, D), Wc, bc.reshape(1, 2),
                           labels.reshape(B, 1))
    return (loss11[0, 0], logits)


# speedup vs baseline: 2.7301x; 2.7301x over previous
import functools

import jax
import jax.numpy as jnp
from jax import lax
from jax.experimental import pallas as pl
from jax.experimental.pallas import tpu as pltpu
from jax.experimental.pallas import tpu_sc as plsc

N = 10000
E = 160000
D = 256
DH = 128
B = 512
EC = 128
NECHUNK = E // EC
PC = 80
NPCHUNK = N // PC
NSUB = 16
RB = 1000

_mesh = plsc.VectorSubcoreMesh(core_axis_name="c", subcore_axis_name="s")
_f32 = jnp.float32


def _mm_first_body(h_ref, wn_ref, wr_ref, b_ref, m0_ref, m1_ref, r_ref):
    h = h_ref[...]
    m = jnp.dot(h, wn_ref[...], preferred_element_type=jnp.float32)
    m0_ref[...] = m[:, :DH]
    m1_ref[...] = m[:, DH:]
    r_ref[...] = jnp.dot(h, wr_ref[...], preferred_element_type=jnp.float32) + b_ref[...]


def _mm_fused_body(a0_ref, a1_ref, rp_ref, wn_ref, wr_ref, b_ref, m0_ref, m1_ref, r_ref):
    h = jnp.maximum(
        jnp.concatenate([a0_ref[...], a1_ref[...]], axis=1) + rp_ref[...], 0.0)
    m = jnp.dot(h, wn_ref[...], preferred_element_type=jnp.float32)
    m0_ref[...] = m[:, :DH]
    m1_ref[...] = m[:, DH:]
    r_ref[...] = jnp.dot(h, wr_ref[...], preferred_element_type=jnp.float32) + b_ref[...]


def _relu_body(a0_ref, a1_ref, rp_ref, h_ref):
    h_ref[...] = jnp.maximum(
        jnp.concatenate([a0_ref[...], a1_ref[...]], axis=1) + rp_ref[...], 0.0)


def _row_spec(w):
    return pl.BlockSpec((RB, w), lambda i: (i, 0))


def _full_spec(a, b):
    return pl.BlockSpec((a, b), lambda i: (0, 0))


_mm_first = pl.pallas_call(
    _mm_first_body,
    grid=(N // RB,),
    in_specs=[_row_spec(D), _full_spec(D, D), _full_spec(D, D), _full_spec(1, D)],
    out_specs=[_row_spec(DH), _row_spec(DH), _row_spec(D)],
    out_shape=[jax.ShapeDtypeStruct((N, DH), _f32),
               jax.ShapeDtypeStruct((N, DH), _f32),
               jax.ShapeDtypeStruct((N, D), _f32)],
)

_mm_fused = pl.pallas_call(
    _mm_fused_body,
    grid=(N // RB,),
    in_specs=[_row_spec(DH), _row_spec(DH), _row_spec(D),
              _full_spec(D, D), _full_spec(D, D), _full_spec(1, D)],
    out_specs=[_row_spec(DH), _row_spec(DH), _row_spec(D)],
    out_shape=[jax.ShapeDtypeStruct((N, DH), _f32),
               jax.ShapeDtypeStruct((N, DH), _f32),
               jax.ShapeDtypeStruct((N, D), _f32)],
)

_relu = pl.pallas_call(
    _relu_body,
    grid=(N // RB,),
    in_specs=[_row_spec(DH), _row_spec(DH), _row_spec(D)],
    out_specs=_row_spec(D),
    out_shape=jax.ShapeDtypeStruct((N, D), _f32),
)


def _head_body(p0s, p1s, cs, rs, p0t, p1t, ct, rt, wlin, blin, wc, bc, lab,
               logits_ref, loss_ref):
    def emb(p0, p1, cnt, root):
        inv = 1.0 / jnp.maximum(cnt[...], 1.0)
        z = jnp.concatenate([p0[...] * inv, p1[...] * inv, root[...]], axis=1)
        return jnp.dot(z, wlin[...], preferred_element_type=jnp.float32) + blin[...]

    u = emb(p0s, p1s, cs, rs)
    v = emb(p0t, p1t, ct, rt)
    feats = jnp.concatenate([u, v, jnp.abs(u - v)], axis=1)
    logits = jnp.dot(feats, wc[...], preferred_element_type=jnp.float32) + bc[...]
    logits_ref[...] = logits
    mx = jnp.max(logits, axis=1, keepdims=True)
    lse = mx + jnp.log(jnp.sum(jnp.exp(logits - mx), axis=1, keepdims=True))
    logp = logits - lse
    sel = jnp.where(lab[...] == 0, logp[:, :1], logp[:, 1:2])
    loss_ref[...] = jnp.full((1, 1), -1.0 / B, jnp.float32) * jnp.sum(sel)


_head = pl.pallas_call(
    _head_body,
    in_specs=[pl.BlockSpec(memory_space=pltpu.VMEM)] * 13,
    out_specs=[pl.BlockSpec(memory_space=pltpu.VMEM)] * 2,
    out_shape=[jax.ShapeDtypeStruct((B, 2), _f32),
               jax.ShapeDtypeStruct((1, 1), _f32)],
)


_ROWS_PER_SUB = N // NSUB


@functools.partial(
    pl.kernel, mesh=_mesh,
    out_type=(jax.ShapeDtypeStruct((N, DH), _f32),
              jax.ShapeDtypeStruct((N, DH), _f32)),
    scratch_types=[
        pltpu.VMEM((EC,), jnp.int32),
        pltpu.VMEM((EC,), jnp.int32),
        pltpu.VMEM((EC, DH), _f32),
        pltpu.VMEM_SHARED((N, DH), _f32),
        pltpu.SemaphoreType.DMA,
    ])
def _scatter(m0_hbm, m1_hbm, src_hbm, dst_hbm, zero_hbm, a0_hbm, a1_hbm,
             idx_s, idx_d, rows, acc, sem):
    c = lax.axis_index("c")
    s = lax.axis_index("s")

    @pl.loop(s, NPCHUNK, step=NSUB)
    def _(k):
        pltpu.sync_copy(zero_hbm.at[pl.ds(k * PC, PC)],
                        acc.at[pl.ds(k * PC, PC)])

    plsc.subcore_barrier()

    def edge_loop(m_hbm):
        @pl.loop(s, NECHUNK, step=NSUB)
        def _(k):
            base = k * EC
            pltpu.sync_copy(src_hbm.at[pl.ds(base, EC)], idx_s)
            pltpu.sync_copy(dst_hbm.at[pl.ds(base, EC)], idx_d)
            pltpu.async_copy(m_hbm.at[idx_s], rows, sem).wait()
            pltpu.sync_copy(rows, acc.at[idx_d], add=True)

    @pl.when(c == 0)
    def _():
        edge_loop(m0_hbm)

    @pl.when(c == 1)
    def _():
        edge_loop(m1_hbm)

    plsc.subcore_barrier()

    @pl.when(c == 0)
    def _():
        @pl.loop(s, NPCHUNK, step=NSUB)
        def _(k):
            pltpu.sync_copy(acc.at[pl.ds(k * PC, PC)],
                            a0_hbm.at[pl.ds(k * PC, PC)])

    @pl.when(c == 1)
    def _():
        @pl.loop(s, NPCHUNK, step=NSUB)
        def _(k):
            pltpu.sync_copy(acc.at[pl.ds(k * PC, PC)],
                            a1_hbm.at[pl.ds(k * PC, PC)])


_BROWS_PER_SUB = B // NSUB


@functools.partial(
    pl.kernel, mesh=_mesh,
    out_type=(jax.ShapeDtypeStruct((B, DH), _f32),
              jax.ShapeDtypeStruct((B, DH), _f32),
              jax.ShapeDtypeStruct((B, DH), _f32),
              jax.ShapeDtypeStruct((B, D), _f32)),
    scratch_types=[
        pltpu.VMEM((PC,), jnp.int32),
        pltpu.VMEM((PC, DH), _f32),
        pltpu.VMEM((PC, DH), _f32),
        pltpu.VMEM((16,), jnp.int32),
        pltpu.VMEM((16, D), _f32),
        pltpu.VMEM_SHARED((B, DH), _f32),
        pltpu.VMEM_SHARED((B, DH), _f32),
        pltpu.SemaphoreType.DMA,
    ])
def _pool(h_hbm, batch_hbm, root_hbm, ones_hbm, zero_hbm,
          p0_hbm, p1_hbm, cnt_hbm, remb_hbm,
          idx_v, rows_v, ones_v, ridx_v, rrows_v, acc, cacc, sem):
    c = lax.axis_index("c")
    s = lax.axis_index("s")
    b0 = s * _BROWS_PER_SUB
    pltpu.sync_copy(zero_hbm.at[pl.ds(b0, _BROWS_PER_SUB)],
                    acc.at[pl.ds(b0, _BROWS_PER_SUB)])
    pltpu.sync_copy(zero_hbm.at[pl.ds(b0, _BROWS_PER_SUB)],
                    cacc.at[pl.ds(b0, _BROWS_PER_SUB)])
    pltpu.sync_copy(ones_hbm, ones_v)
    plsc.subcore_barrier()

    w = s * 2 + c
    pltpu.sync_copy(root_hbm.at[pl.ds(w * 16, 16)], ridx_v)
    pltpu.async_copy(h_hbm.at[ridx_v], rrows_v, sem).wait()
    pltpu.sync_copy(rrows_v, remb_hbm.at[pl.ds(w * 16, 16)])

    def pool_loop(col):
        @pl.loop(s, NPCHUNK, step=NSUB)
        def _(k):
            base = k * PC
            pltpu.sync_copy(batch_hbm.at[pl.ds(base, PC)], idx_v)
            pltpu.sync_copy(h_hbm.at[pl.ds(base, PC), pl.ds(col, DH)], rows_v)
            pltpu.sync_copy(rows_v, acc.at[idx_v], add=True)
            pltpu.sync_copy(ones_v, cacc.at[idx_v], add=True)

    @pl.when(c == 0)
    def _():
        pool_loop(0)

    @pl.when(c == 1)
    def _():
        pool_loop(DH)

    plsc.subcore_barrier()

    @pl.when(c == 0)
    def _():
        pltpu.sync_copy(acc.at[pl.ds(b0, _BROWS_PER_SUB)],
                        p0_hbm.at[pl.ds(b0, _BROWS_PER_SUB)])
        pltpu.sync_copy(cacc.at[pl.ds(b0, _BROWS_PER_SUB)],
                        cnt_hbm.at[pl.ds(b0, _BROWS_PER_SUB)])

    @pl.when(c == 1)
    def _():
        pltpu.sync_copy(acc.at[pl.ds(b0, _BROWS_PER_SUB)],
                        p1_hbm.at[pl.ds(b0, _BROWS_PER_SUB)])


def kernel(x_s, edge_index_s, batch_s, root_n_id_s, x_t, edge_index_t,
           batch_t, root_n_id_t, labels, Wr1, Wn1, b1, Wr2, Wn2, b2,
           Wr3, Wn3, b3, Wlin, blin, Wc, bc):
    zeros = jnp.zeros((N, DH), _f32)
    ones = jnp.ones((PC, DH), _f32)
    b1r = b1.reshape(1, D)
    b2r = b2.reshape(1, D)
    b3r = b3.reshape(1, D)

    def gnn(x, ei, batch, root):
        src, dst = ei[0], ei[1]
        m0, m1, r = _mm_first(x, Wn1, Wr1, b1r)
        a0, a1 = _scatter(m0, m1, src, dst, zeros)
        m0, m1, r = _mm_fused(a0, a1, r, Wn2, Wr2, b2r)
        a0, a1 = _scatter(m0, m1, src, dst, zeros)
        m0, m1, r = _mm_fused(a0, a1, r, Wn3, Wr3, b3r)
        a0, a1 = _scatter(m0, m1, src, dst, zeros)
        h3 = _relu(a0, a1, r)
        return _pool(h3, batch, root, ones, zeros)

    p0s, p1s, cs, rs = gnn(x_s, edge_index_s, batch_s, root_n_id_s)
    p0t, p1t, ct, rt = gnn(x_t, edge_index_t, batch_t, root_n_id_t)

    logits, loss11 = _head(p0s, p1s, cs, rs, p0t, p1t, ct, rt,
                           Wlin, blin.reshape(1, D), Wc, bc.reshape(1, 2),
                           labels.reshape(B, 1))
    return (loss11[0, 0], logits)
